# baseline (speedup 1.0000x reference)
import functools

import jax
import jax.numpy as jnp
from jax import lax
from jax.experimental import pallas as pl
from jax.experimental.pallas import tpu as pltpu
from jax.experimental.pallas import tpu_sc as plsc

NC = 2
NS = 16
L = 16
NW = NC * NS
B = 16384
BPW = B // NW
CH = 128
NCHUNK = BPW // CH
D = 64
NGROUP = BPW // L
GBLK = 1_000_000 // L


def _rsqrt(x):
    xi = plsc.bitcast(x, jnp.int32)
    yi = jnp.int32(0x5F3759DF) - (xi >> 1)
    y = plsc.bitcast(yi, jnp.float32)
    xh = x * 0.5
    for _ in range(3):
        y = y * (1.5 - xh * y * y)
    return y


_mesh = plsc.VectorSubcoreMesh(core_axis_name="c", subcore_axis_name="s")


def _bpr_body(u3, i3, j3, vu_hbm, gu_hbm, vv_hbm, gv_hbm, out_hbm,
              idx_u, idx_i, idx_j, blk_u, blk_i, blk_j,
              wu_v, hi_v, hj_v, gu_v, gi_v, gj_v, obuf, sem):
    wid = lax.axis_index("s") * NC + lax.axis_index("c")

    pltpu.sync_copy(u3.at[wid], idx_u)
    pltpu.sync_copy(i3.at[wid], idx_i)
    pltpu.sync_copy(j3.at[wid], idx_j)

    for k in range(NCHUNK):
        for c in range(CH // L):
            sl = pl.ds(c * L, L)
            blk_u[k, sl] = idx_u[k, sl] >> 4
            blk_i[k, sl] = idx_i[k, sl] >> 4
            blk_j[k, sl] = idx_j[k, sl] >> 4

    descs = []
    for k in range(NCHUNK):
        sl = pl.ds(k * CH, CH)
        descs.append(pltpu.async_copy(vu_hbm.at[idx_u.at[k]], wu_v.at[sl], sem))
        descs.append(pltpu.async_copy(vv_hbm.at[idx_i.at[k]], hi_v.at[sl], sem))
        descs.append(pltpu.async_copy(vv_hbm.at[idx_j.at[k]], hj_v.at[sl], sem))
        descs.append(pltpu.async_copy(gu_hbm.at[blk_u.at[k]], gu_v.at[sl], sem))
        descs.append(pltpu.async_copy(gv_hbm.at[blk_i.at[k]], gi_v.at[sl], sem))
        descs.append(pltpu.async_copy(gv_hbm.at[blk_j.at[k]], gj_v.at[sl], sem))
    for dsc in descs:
        dsc.wait()

    lanes = lax.iota(jnp.int32, L)
    zf = jnp.zeros((L,), jnp.float32)
    m15 = jnp.full((L,), 15, jnp.int32)

    def group_body(g, accs):
        pi_acc, pj_acc = accs
        rows = g * L + lanes
        chunk = jnp.full((L,), 0, jnp.int32) + (g >> 3)
        icol = (g * L) % CH + lanes
        iu = plsc.load_gather(idx_u, [chunk, icol])
        ii = plsc.load_gather(idx_i, [chunk, icol])
        ij = plsc.load_gather(idx_j, [chunk, icol])

        def d_body(dd, carry):
            ssu, ssi, ssj, dui, duj = carry
            col = jnp.full((L,), 0, jnp.int32) + dd
            wu = plsc.load_gather(wu_v, [rows, col])
            hi = plsc.load_gather(hi_v, [rows, col])
            hj = plsc.load_gather(hj_v, [rows, col])
            return (ssu + wu * wu, ssi + hi * hi, ssj + hj * hj,
                    dui + wu * hi, duj + wu * hj)

        ssu, ssi, ssj, dui, duj = lax.fori_loop(
            0, D, d_body, (zf, zf, zf, zf, zf))

        gu = plsc.load_gather(gu_v, [rows, iu & m15])
        gi = plsc.load_gather(gi_v, [rows, ii & m15])
        gj = plsc.load_gather(gj_v, [rows, ij & m15])
        su = gu * _rsqrt(ssu)
        pi_acc = pi_acc + (su * gi * _rsqrt(ssi)) * dui
        pj_acc = pj_acc + (su * gj * _rsqrt(ssj)) * duj
        return (pi_acc, pj_acc)

    pi_acc, pj_acc = lax.fori_loop(0, NGROUP, group_body, (zf, zf))
    pi_s = jnp.sum(pi_acc)
    pj_s = jnp.sum(pj_acc)
    obuf[...] = jnp.where(lanes == 0, pi_s,
                          jnp.where(lanes == 1, pj_s, 0.0))
    pltpu.sync_copy(obuf, out_hbm.at[wid])


_SCRATCH = [
    pltpu.VMEM((NCHUNK, CH), jnp.int32),
    pltpu.VMEM((NCHUNK, CH), jnp.int32),
    pltpu.VMEM((NCHUNK, CH), jnp.int32),
    pltpu.VMEM((NCHUNK, CH), jnp.int32),
    pltpu.VMEM((NCHUNK, CH), jnp.int32),
    pltpu.VMEM((NCHUNK, CH), jnp.int32),
    pltpu.VMEM((BPW, D), jnp.float32),
    pltpu.VMEM((BPW, D), jnp.float32),
    pltpu.VMEM((BPW, D), jnp.float32),
    pltpu.VMEM((BPW, L), jnp.float32),
    pltpu.VMEM((BPW, L), jnp.float32),
    pltpu.VMEM((BPW, L), jnp.float32),
    pltpu.VMEM((L,), jnp.float32),
    pltpu.SemaphoreType.DMA,
]

_bpr_sc = functools.partial(
    pl.kernel,
    out_type=jax.ShapeDtypeStruct((NW, L), jnp.float32),
    mesh=_mesh,
    compiler_params=pltpu.CompilerParams(
        needs_layout_passes=False, use_tc_tiling_on_sc=False),
    scratch_types=_SCRATCH,
)(_bpr_body)


def kernel(u, i, j, v_u, g_u, v_v, g_v):
    u3 = u.astype(jnp.int32).reshape(NW, NCHUNK, CH)
    i3 = i.astype(jnp.int32).reshape(NW, NCHUNK, CH)
    j3 = j.astype(jnp.int32).reshape(NW, NCHUNK, CH)
    gu_blk = g_u.reshape(GBLK, L)
    gv_blk = g_v.reshape(GBLK, L)
    partials = _bpr_sc(u3, i3, j3, v_u, gu_blk, v_v, gv_blk)
    return (jnp.sum(partials[:, 0]), jnp.sum(partials[:, 1]))
